# Initial kernel scaffold; baseline (speedup 1.0000x reference)
import jax
import jax.numpy as jnp
from jax.experimental import pallas as pl


def kernel(x, edge_index, batch, W1, b1, W2, b2, Wa, ba, Wg1, bg1, Wg2, bg2, Wc1, bc1, Wc2, bc2):
    raise NotImplementedError("write your pallas kernel here")


# speedup vs baseline: 6.8732x; 6.8732x over previous
import functools

import jax
import jax.numpy as jnp
from jax import lax
from jax.experimental import pallas as pl
from jax.experimental.pallas import tpu as pltpu
from jax.experimental.pallas import tpu_sc as plsc

N = 10000
NP = 10240
E = 160000
EP = 163840
NG = 64
NTILE = 16
EPT = EP // NTILE
NROW = NP // NTILE
CHUNK = 256
N_CH = EPT // CHUNK
HALF = 128

_SC_MESH = dict(core_axis_name="c", subcore_axis_name="s")


def _make_deg():
    mesh = plsc.VectorSubcoreMesh(**_SC_MESH)

    @functools.partial(
        pl.kernel, mesh=mesh,
        out_type=jax.ShapeDtypeStruct((NP,), jnp.float32),
        scratch_types=[
            pltpu.VMEM_SHARED((NP,), jnp.float32),
            pltpu.VMEM((EPT,), jnp.int32),
            pltpu.VMEM((EPT,), jnp.float32),
            pltpu.VMEM((NROW,), jnp.float32),
        ],
    )
    def deg_k(dst_hbm, ones_hbm, deg_out, acc_sh, dst_v, ones_v, blk_v):
        c = lax.axis_index("c")
        s = lax.axis_index("s")
        pltpu.sync_copy(dst_hbm.at[pl.ds(s * EPT, EPT)], dst_v)
        pltpu.sync_copy(ones_hbm, ones_v)

        def zero_blk(i, carry):
            blk_v[pl.ds(i * 16, 16)] = jnp.zeros((16,), jnp.float32)
            return carry
        lax.fori_loop(0, NROW // 16, zero_blk, 0)
        pltpu.sync_copy(blk_v, acc_sh.at[pl.ds(s * NROW, NROW)])
        plsc.subcore_barrier()
        pltpu.sync_copy(ones_v, acc_sh.at[dst_v], add=True)
        plsc.subcore_barrier()

        @pl.when(c == 0)
        def _():
            pltpu.sync_copy(acc_sh.at[pl.ds(s * NROW, NROW)],
                            deg_out.at[pl.ds(s * NROW, NROW)])

    return deg_k


def _make_agg():
    mesh = plsc.VectorSubcoreMesh(**_SC_MESH)

    @functools.partial(
        pl.kernel, mesh=mesh,
        out_type=(jax.ShapeDtypeStruct((NP, HALF), jnp.float32),
                  jax.ShapeDtypeStruct((NP, HALF), jnp.float32)),
        scratch_types=[
            pltpu.VMEM_SHARED((NP, HALF), jnp.float32),
            pltpu.VMEM((CHUNK,), jnp.int32),
            pltpu.VMEM((CHUNK,), jnp.int32),
            pltpu.VMEM((CHUNK, HALF), jnp.float32),
            pltpu.SemaphoreType.DMA,
        ],
    )
    def agg_k(lo_hbm, hi_hbm, src_hbm, dst_hbm, out_lo, out_hi,
              acc_sh, srcc_v, dstc_v, rows_v, sem):
        c = lax.axis_index("c")
        s = lax.axis_index("s")

        def run(hw_hbm, out_hbm):
            pltpu.sync_copy(hw_hbm.at[pl.ds(s * NROW, NROW)],
                            acc_sh.at[pl.ds(s * NROW, NROW)])
            plsc.subcore_barrier()

            def body(j, carry):
                base = s * EPT + j * CHUNK
                pltpu.sync_copy(src_hbm.at[pl.ds(base, CHUNK)], srcc_v)
                pltpu.sync_copy(dst_hbm.at[pl.ds(base, CHUNK)], dstc_v)
                pltpu.async_copy(hw_hbm.at[srcc_v], rows_v, sem).wait()
                pltpu.sync_copy(rows_v, acc_sh.at[dstc_v], add=True)
                return carry
            lax.fori_loop(0, N_CH, body, 0)
            plsc.subcore_barrier()
            pltpu.sync_copy(acc_sh.at[pl.ds(s * NROW, NROW)],
                            out_hbm.at[pl.ds(s * NROW, NROW)])

        @pl.when(c == 0)
        def _():
            run(lo_hbm, out_lo)

        @pl.when(c == 1)
        def _():
            run(hi_hbm, out_hi)

    return agg_k


_BLK = 512
_BLK3 = 400


def _tc1_body(x_ref, w1_ref, b1_ref, b2_ref, wa_ref, ba_ref, wg1_ref,
              deg_ref, lo_ref, hi_ref, dinv_ref):
    xb = x_ref[...]
    h1 = lax.dot_general(xb, w1_ref[...], (((1,), (1,)), ((), ())),
                         preferred_element_type=jnp.float32) + b1_ref[...]
    a1 = jnp.sum(h1 * wa_ref[...], axis=1, keepdims=True)
    a2 = jnp.sum(b2_ref[...] * wa_ref[...])
    w0 = jax.nn.sigmoid(a1 - a2)
    fused = w0 * h1 + (1.0 - w0) * b2_ref[...]
    hw = lax.dot_general(fused, wg1_ref[...], (((1,), (1,)), ((), ())),
                         preferred_element_type=jnp.float32)
    dinv = lax.rsqrt(deg_ref[...] + 1.0)
    hws = hw * dinv
    lo_ref[...] = hws[:, :HALF]
    hi_ref[...] = hws[:, HALF:]
    dinv_ref[...] = dinv


def _tc1(x_pad, W1, b1r, b2r, Wa, bar, Wg1, deg2):
    grid = NP // _BLK
    return pl.pallas_call(
        _tc1_body,
        grid=(grid,),
        in_specs=[
            pl.BlockSpec((_BLK, 768), lambda i: (i, 0)),
            pl.BlockSpec((768, 768), lambda i: (0, 0)),
            pl.BlockSpec((1, 768), lambda i: (0, 0)),
            pl.BlockSpec((1, 768), lambda i: (0, 0)),
            pl.BlockSpec((1, 768), lambda i: (0, 0)),
            pl.BlockSpec((1, 1), lambda i: (0, 0)),
            pl.BlockSpec((256, 768), lambda i: (0, 0)),
            pl.BlockSpec((_BLK, 1), lambda i: (i, 0)),
        ],
        out_specs=[
            pl.BlockSpec((_BLK, HALF), lambda i: (i, 0)),
            pl.BlockSpec((_BLK, HALF), lambda i: (i, 0)),
            pl.BlockSpec((_BLK, 1), lambda i: (i, 0)),
        ],
        out_shape=[
            jax.ShapeDtypeStruct((NP, HALF), jnp.float32),
            jax.ShapeDtypeStruct((NP, HALF), jnp.float32),
            jax.ShapeDtypeStruct((NP, 1), jnp.float32),
        ],
    )(x_pad, W1, b1r, b2r, Wa, bar, Wg1, deg2)


def _tc2_body(lo_ref, hi_ref, dinv_ref, bg1_ref, wg2_ref, lo_o, hi_o):
    acc = jnp.concatenate([lo_ref[...], hi_ref[...]], axis=1)
    dinv = dinv_ref[...]
    h = jnp.maximum(acc * dinv + bg1_ref[...], 0.0)
    hw = lax.dot_general(h, wg2_ref[...], (((1,), (1,)), ((), ())),
                         preferred_element_type=jnp.float32) * dinv
    lo_o[...] = hw[:, :HALF]
    hi_o[...] = hw[:, HALF:]


def _tc2(acc_lo, acc_hi, dinv2, bg1r, Wg2):
    grid = NP // _BLK
    return pl.pallas_call(
        _tc2_body,
        grid=(grid,),
        in_specs=[
            pl.BlockSpec((_BLK, HALF), lambda i: (i, 0)),
            pl.BlockSpec((_BLK, HALF), lambda i: (i, 0)),
            pl.BlockSpec((_BLK, 1), lambda i: (i, 0)),
            pl.BlockSpec((1, 256), lambda i: (0, 0)),
            pl.BlockSpec((256, 256), lambda i: (0, 0)),
        ],
        out_specs=[
            pl.BlockSpec((_BLK, HALF), lambda i: (i, 0)),
            pl.BlockSpec((_BLK, HALF), lambda i: (i, 0)),
        ],
        out_shape=[
            jax.ShapeDtypeStruct((NP, HALF), jnp.float32),
            jax.ShapeDtypeStruct((NP, HALF), jnp.float32),
        ],
    )(acc_lo, acc_hi, dinv2, bg1r, Wg2)


def _tc3_body(lo_ref, hi_ref, dinv_ref, bg2_ref, batch_ref, wc1_ref, bc1_ref,
              wc2_ref, bc2_ref, out_ref, psum, pcnt):
    i = pl.program_id(0)

    @pl.when(i == 0)
    def _():
        psum[...] = jnp.zeros_like(psum)
        pcnt[...] = jnp.zeros_like(pcnt)

    acc = jnp.concatenate([lo_ref[...], hi_ref[...]], axis=1)
    h = jnp.maximum(acc * dinv_ref[...] + bg2_ref[...], 0.0)
    b = batch_ref[0]
    oh = (lax.broadcasted_iota(jnp.int32, (NG, _BLK3), 0) == b
          ).astype(jnp.float32)
    psum[...] += lax.dot_general(oh, h, (((1,), (0,)), ((), ())),
                                 preferred_element_type=jnp.float32)
    pcnt[...] += jnp.sum(oh, axis=1, keepdims=True)

    @pl.when(i == (N // _BLK3) - 1)
    def _():
        pooled = psum[...] / jnp.maximum(pcnt[...], 1.0)
        z = jnp.maximum(
            lax.dot_general(pooled, wc1_ref[...], (((1,), (1,)), ((), ())),
                            preferred_element_type=jnp.float32) + bc1_ref[...],
            0.0)
        out_ref[...] = lax.dot_general(
            z, wc2_ref[...], (((1,), (1,)), ((), ())),
            preferred_element_type=jnp.float32) + bc2_ref[...]


def _tc3(acc_lo, acc_hi, dinv2, bg2r, batch3, Wc1, bc1r, Wc2, bc2r):
    grid = N // _BLK3
    return pl.pallas_call(
        _tc3_body,
        grid=(grid,),
        in_specs=[
            pl.BlockSpec((_BLK3, HALF), lambda i: (i, 0)),
            pl.BlockSpec((_BLK3, HALF), lambda i: (i, 0)),
            pl.BlockSpec((_BLK3, 1), lambda i: (i, 0)),
            pl.BlockSpec((1, 256), lambda i: (0, 0)),
            pl.BlockSpec((1, 1, _BLK3), lambda i: (i, 0, 0)),
            pl.BlockSpec((128, 256), lambda i: (0, 0)),
            pl.BlockSpec((1, 128), lambda i: (0, 0)),
            pl.BlockSpec((5, 128), lambda i: (0, 0)),
            pl.BlockSpec((1, 5), lambda i: (0, 0)),
        ],
        out_specs=pl.BlockSpec((NG, 5), lambda i: (0, 0)),
        out_shape=jax.ShapeDtypeStruct((NG, 5), jnp.float32),
        scratch_shapes=[
            pltpu.VMEM((NG, 256), jnp.float32),
            pltpu.VMEM((NG, 1), jnp.float32),
        ],
    )(acc_lo, acc_hi, dinv2, bg2r, batch3, Wc1, bc1r, Wc2, bc2r)


_deg_k = _make_deg()
_agg_k = _make_agg()


def kernel(x, edge_index, batch, W1, b1, W2, b2, Wa, ba, Wg1, bg1, Wg2, bg2,
           Wc1, bc1, Wc2, bc2):
    del W2
    f32 = jnp.float32
    src = edge_index[0]
    dst = edge_index[1]
    pad_e = jnp.full((EP - E,), N, jnp.int32)
    srcp = jnp.concatenate([src, pad_e])
    dstp = jnp.concatenate([dst, pad_e])
    ones = jnp.ones((EPT,), f32)
    x_pad = jnp.concatenate([x, jnp.zeros((NP - N, 768), f32)])
    b1r = b1.reshape(1, 768)
    b2r = b2.reshape(1, 768)
    bar = ba.reshape(1, 1)
    bg1r = bg1.reshape(1, 256)
    bg2r = bg2.reshape(1, 256)
    bc1r = bc1.reshape(1, 128)
    bc2r = bc2.reshape(1, 5)
    batch3 = batch.reshape(N // _BLK3, 1, _BLK3)

    deg = _deg_k(dstp, ones)
    deg2 = deg.reshape(NP, 1)
    hw_lo, hw_hi, dinv2 = _tc1(x_pad, W1, b1r, b2r, Wa, bar, Wg1, deg2)
    a1_lo, a1_hi = _agg_k(hw_lo, hw_hi, srcp, dstp)
    hw2_lo, hw2_hi = _tc2(a1_lo, a1_hi, dinv2, bg1r, Wg2)
    a2_lo, a2_hi = _agg_k(hw2_lo, hw2_hi, srcp, dstp)
    return _tc3(a2_lo, a2_hi, dinv2, bg2r, batch3, Wc1, bc1r, Wc2, bc2r)


# speedup vs baseline: 7.8836x; 1.1470x over previous
import functools

import jax
import jax.numpy as jnp
from jax import lax
from jax.experimental import pallas as pl
from jax.experimental.pallas import tpu as pltpu
from jax.experimental.pallas import tpu_sc as plsc

N = 10000
NP = 10240
E = 160000
EP = 163840
NG = 64
NTILE = 16
EPT = EP // NTILE
NROW = NP // NTILE
CHUNK = 160
N_CH = EPT // CHUNK
HALF = 128

_SC_MESH = dict(core_axis_name="c", subcore_axis_name="s")


def _make_deg():
    mesh = plsc.VectorSubcoreMesh(**_SC_MESH)

    @functools.partial(
        pl.kernel, mesh=mesh,
        out_type=jax.ShapeDtypeStruct((NP,), jnp.float32),
        scratch_types=[
            pltpu.VMEM_SHARED((NP,), jnp.float32),
            pltpu.VMEM((EPT,), jnp.int32),
            pltpu.VMEM((EPT,), jnp.float32),
            pltpu.VMEM((NROW,), jnp.float32),
        ],
    )
    def deg_k(dst_hbm, ones_hbm, deg_out, acc_sh, dst_v, ones_v, blk_v):
        c = lax.axis_index("c")
        s = lax.axis_index("s")
        pltpu.sync_copy(dst_hbm.at[pl.ds(s * EPT, EPT)], dst_v)
        pltpu.sync_copy(ones_hbm, ones_v)

        def zero_blk(i, carry):
            blk_v[pl.ds(i * 16, 16)] = jnp.zeros((16,), jnp.float32)
            return carry
        lax.fori_loop(0, NROW // 16, zero_blk, 0)
        pltpu.sync_copy(blk_v, acc_sh.at[pl.ds(s * NROW, NROW)])
        plsc.subcore_barrier()
        pltpu.sync_copy(ones_v, acc_sh.at[dst_v], add=True)
        plsc.subcore_barrier()

        @pl.when(c == 0)
        def _():
            pltpu.sync_copy(acc_sh.at[pl.ds(s * NROW, NROW)],
                            deg_out.at[pl.ds(s * NROW, NROW)])

    return deg_k


def _make_agg():
    mesh = plsc.VectorSubcoreMesh(**_SC_MESH)

    @functools.partial(
        pl.kernel, mesh=mesh,
        out_type=(jax.ShapeDtypeStruct((NP, HALF), jnp.float32),
                  jax.ShapeDtypeStruct((NP, HALF), jnp.float32)),
        scratch_types=[
            pltpu.VMEM_SHARED((NP, HALF), jnp.float32),
            pltpu.VMEM((CHUNK,), jnp.int32),
            pltpu.VMEM((CHUNK,), jnp.int32),
            pltpu.VMEM((CHUNK,), jnp.int32),
            pltpu.VMEM((CHUNK,), jnp.int32),
            pltpu.VMEM((CHUNK, HALF), jnp.float32),
            pltpu.VMEM((CHUNK, HALF), jnp.float32),
            pltpu.SemaphoreType.DMA,
            pltpu.SemaphoreType.DMA,
            pltpu.SemaphoreType.DMA,
            pltpu.SemaphoreType.DMA,
            pltpu.SemaphoreType.DMA,
            pltpu.SemaphoreType.DMA,
        ],
    )
    def agg_k(lo_hbm, hi_hbm, src_hbm, dst_hbm, out_lo, out_hi,
              acc_sh, srcc0, srcc1, dstc0, dstc1, rows0, rows1,
              si0, si1, sg0, sg1, ss0, ss1):
        c = lax.axis_index("c")
        s = lax.axis_index("s")
        slots = ((srcc0, dstc0, rows0, si0, sg0, ss0),
                 (srcc1, dstc1, rows1, si1, sg1, ss1))

        def run(hw_hbm, out_hbm):
            pltpu.sync_copy(hw_hbm.at[pl.ds(s * NROW, NROW)],
                            acc_sh.at[pl.ds(s * NROW, NROW)])
            plsc.subcore_barrier()
            base = s * EPT

            def issue_idx(k, slot):
                sc, dc, _, si, _, _ = slot
                off = pl.multiple_of(base + k * CHUNK, CHUNK)
                pltpu.async_copy(src_hbm.at[pl.ds(off, CHUNK)], sc, si)
                pltpu.async_copy(dst_hbm.at[pl.ds(off, CHUNK)], dc, si)

            def wait_idx(slot):
                sc, dc, _, si, _, _ = slot
                off = pl.multiple_of(base, CHUNK)
                pltpu.make_async_copy(src_hbm.at[pl.ds(off, CHUNK)], sc, si).wait()
                pltpu.make_async_copy(dst_hbm.at[pl.ds(off, CHUNK)], dc, si).wait()

            issue_idx(0, slots[0])
            wait_idx(slots[0])
            pltpu.async_copy(hw_hbm.at[slots[0][0]], slots[0][2], slots[0][4])

            def body2(i, carry):
                for b in (0, 1):
                    k = 2 * i + b
                    sc, dc, rv, si, sg, ss = slots[b]
                    scn, dcn, rvn, sin, sgn, ssn = slots[1 - b]

                    @pl.when(k >= 1)
                    def _():
                        pltpu.make_async_copy(
                            rvn, acc_sh.at[dcn], ssn).wait()

                    @pl.when(k + 1 < N_CH)
                    def _():
                        issue_idx(k + 1, slots[1 - b])

                    pltpu.make_async_copy(hw_hbm.at[sc], rv, sg).wait()

                    @pl.when(k + 1 < N_CH)
                    def _():
                        wait_idx(slots[1 - b])
                        pltpu.async_copy(hw_hbm.at[scn], rvn, sgn)

                    pltpu.async_copy(rv, acc_sh.at[dc], ss, add=True)
                return carry
            lax.fori_loop(0, N_CH // 2, body2, 0)
            pltpu.make_async_copy(rows1, acc_sh.at[dstc1], ss1).wait()
            plsc.subcore_barrier()
            pltpu.sync_copy(acc_sh.at[pl.ds(s * NROW, NROW)],
                            out_hbm.at[pl.ds(s * NROW, NROW)])

        @pl.when(c == 0)
        def _():
            run(lo_hbm, out_lo)

        @pl.when(c == 1)
        def _():
            run(hi_hbm, out_hi)

    return agg_k


_BLK = 512
_BLK3 = 400


def _tc1_body(x_ref, w1_ref, b1_ref, b2_ref, wa_ref, ba_ref, wg1_ref,
              deg_ref, lo_ref, hi_ref, dinv_ref):
    xb = x_ref[...]
    h1 = lax.dot_general(xb, w1_ref[...], (((1,), (1,)), ((), ())),
                         preferred_element_type=jnp.float32) + b1_ref[...]
    a1 = jnp.sum(h1 * wa_ref[...], axis=1, keepdims=True)
    a2 = jnp.sum(b2_ref[...] * wa_ref[...])
    w0 = jax.nn.sigmoid(a1 - a2)
    fused = w0 * h1 + (1.0 - w0) * b2_ref[...]
    hw = lax.dot_general(fused, wg1_ref[...], (((1,), (1,)), ((), ())),
                         preferred_element_type=jnp.float32)
    dinv = lax.rsqrt(deg_ref[...] + 1.0)
    hws = hw * dinv
    lo_ref[...] = hws[:, :HALF]
    hi_ref[...] = hws[:, HALF:]
    dinv_ref[...] = dinv


def _tc1(x_pad, W1, b1r, b2r, Wa, bar, Wg1, deg2):
    grid = NP // _BLK
    return pl.pallas_call(
        _tc1_body,
        grid=(grid,),
        in_specs=[
            pl.BlockSpec((_BLK, 768), lambda i: (i, 0)),
            pl.BlockSpec((768, 768), lambda i: (0, 0)),
            pl.BlockSpec((1, 768), lambda i: (0, 0)),
            pl.BlockSpec((1, 768), lambda i: (0, 0)),
            pl.BlockSpec((1, 768), lambda i: (0, 0)),
            pl.BlockSpec((1, 1), lambda i: (0, 0)),
            pl.BlockSpec((256, 768), lambda i: (0, 0)),
            pl.BlockSpec((_BLK, 1), lambda i: (i, 0)),
        ],
        out_specs=[
            pl.BlockSpec((_BLK, HALF), lambda i: (i, 0)),
            pl.BlockSpec((_BLK, HALF), lambda i: (i, 0)),
            pl.BlockSpec((_BLK, 1), lambda i: (i, 0)),
        ],
        out_shape=[
            jax.ShapeDtypeStruct((NP, HALF), jnp.float32),
            jax.ShapeDtypeStruct((NP, HALF), jnp.float32),
            jax.ShapeDtypeStruct((NP, 1), jnp.float32),
        ],
    )(x_pad, W1, b1r, b2r, Wa, bar, Wg1, deg2)


def _tc2_body(lo_ref, hi_ref, dinv_ref, bg1_ref, wg2_ref, lo_o, hi_o):
    acc = jnp.concatenate([lo_ref[...], hi_ref[...]], axis=1)
    dinv = dinv_ref[...]
    h = jnp.maximum(acc * dinv + bg1_ref[...], 0.0)
    hw = lax.dot_general(h, wg2_ref[...], (((1,), (1,)), ((), ())),
                         preferred_element_type=jnp.float32) * dinv
    lo_o[...] = hw[:, :HALF]
    hi_o[...] = hw[:, HALF:]


def _tc2(acc_lo, acc_hi, dinv2, bg1r, Wg2):
    grid = NP // _BLK
    return pl.pallas_call(
        _tc2_body,
        grid=(grid,),
        in_specs=[
            pl.BlockSpec((_BLK, HALF), lambda i: (i, 0)),
            pl.BlockSpec((_BLK, HALF), lambda i: (i, 0)),
            pl.BlockSpec((_BLK, 1), lambda i: (i, 0)),
            pl.BlockSpec((1, 256), lambda i: (0, 0)),
            pl.BlockSpec((256, 256), lambda i: (0, 0)),
        ],
        out_specs=[
            pl.BlockSpec((_BLK, HALF), lambda i: (i, 0)),
            pl.BlockSpec((_BLK, HALF), lambda i: (i, 0)),
        ],
        out_shape=[
            jax.ShapeDtypeStruct((NP, HALF), jnp.float32),
            jax.ShapeDtypeStruct((NP, HALF), jnp.float32),
        ],
    )(acc_lo, acc_hi, dinv2, bg1r, Wg2)


def _tc3_body(lo_ref, hi_ref, dinv_ref, bg2_ref, batch_ref, wc1_ref, bc1_ref,
              wc2_ref, bc2_ref, out_ref, psum, pcnt):
    i = pl.program_id(0)

    @pl.when(i == 0)
    def _():
        psum[...] = jnp.zeros_like(psum)
        pcnt[...] = jnp.zeros_like(pcnt)

    acc = jnp.concatenate([lo_ref[...], hi_ref[...]], axis=1)
    h = jnp.maximum(acc * dinv_ref[...] + bg2_ref[...], 0.0)
    b = batch_ref[0]
    oh = (lax.broadcasted_iota(jnp.int32, (NG, _BLK3), 0) == b
          ).astype(jnp.float32)
    psum[...] += lax.dot_general(oh, h, (((1,), (0,)), ((), ())),
                                 preferred_element_type=jnp.float32)
    pcnt[...] += jnp.sum(oh, axis=1, keepdims=True)

    @pl.when(i == (N // _BLK3) - 1)
    def _():
        pooled = psum[...] / jnp.maximum(pcnt[...], 1.0)
        z = jnp.maximum(
            lax.dot_general(pooled, wc1_ref[...], (((1,), (1,)), ((), ())),
                            preferred_element_type=jnp.float32) + bc1_ref[...],
            0.0)
        out_ref[...] = lax.dot_general(
            z, wc2_ref[...], (((1,), (1,)), ((), ())),
            preferred_element_type=jnp.float32) + bc2_ref[...]


def _tc3(acc_lo, acc_hi, dinv2, bg2r, batch3, Wc1, bc1r, Wc2, bc2r):
    grid = N // _BLK3
    return pl.pallas_call(
        _tc3_body,
        grid=(grid,),
        in_specs=[
            pl.BlockSpec((_BLK3, HALF), lambda i: (i, 0)),
            pl.BlockSpec((_BLK3, HALF), lambda i: (i, 0)),
            pl.BlockSpec((_BLK3, 1), lambda i: (i, 0)),
            pl.BlockSpec((1, 256), lambda i: (0, 0)),
            pl.BlockSpec((1, 1, _BLK3), lambda i: (i, 0, 0)),
            pl.BlockSpec((128, 256), lambda i: (0, 0)),
            pl.BlockSpec((1, 128), lambda i: (0, 0)),
            pl.BlockSpec((5, 128), lambda i: (0, 0)),
            pl.BlockSpec((1, 5), lambda i: (0, 0)),
        ],
        out_specs=pl.BlockSpec((NG, 5), lambda i: (0, 0)),
        out_shape=jax.ShapeDtypeStruct((NG, 5), jnp.float32),
        scratch_shapes=[
            pltpu.VMEM((NG, 256), jnp.float32),
            pltpu.VMEM((NG, 1), jnp.float32),
        ],
    )(acc_lo, acc_hi, dinv2, bg2r, batch3, Wc1, bc1r, Wc2, bc2r)


_deg_k = _make_deg()
_agg_k = _make_agg()


def kernel(x, edge_index, batch, W1, b1, W2, b2, Wa, ba, Wg1, bg1, Wg2, bg2,
           Wc1, bc1, Wc2, bc2):
    del W2
    f32 = jnp.float32
    src = edge_index[0]
    dst = edge_index[1]
    pad_e = jnp.full((EP - E,), N, jnp.int32)
    srcp = jnp.concatenate([src, pad_e])
    dstp = jnp.concatenate([dst, pad_e])
    ones = jnp.ones((EPT,), f32)
    x_pad = jnp.concatenate([x, jnp.zeros((NP - N, 768), f32)])
    b1r = b1.reshape(1, 768)
    b2r = b2.reshape(1, 768)
    bar = ba.reshape(1, 1)
    bg1r = bg1.reshape(1, 256)
    bg2r = bg2.reshape(1, 256)
    bc1r = bc1.reshape(1, 128)
    bc2r = bc2.reshape(1, 5)
    batch3 = batch.reshape(N // _BLK3, 1, _BLK3)

    deg = _deg_k(dstp, ones)
    deg2 = deg.reshape(NP, 1)
    hw_lo, hw_hi, dinv2 = _tc1(x_pad, W1, b1r, b2r, Wa, bar, Wg1, deg2)
    a1_lo, a1_hi = _agg_k(hw_lo, hw_hi, srcp, dstp)
    hw2_lo, hw2_hi = _tc2(a1_lo, a1_hi, dinv2, bg1r, Wg2)
    a2_lo, a2_hi = _agg_k(hw2_lo, hw2_hi, srcp, dstp)
    return _tc3(a2_lo, a2_hi, dinv2, bg2r, batch3, Wc1, bc1r, Wc2, bc2r)


# speedup vs baseline: 8.3840x; 1.0635x over previous
import functools

import jax
import jax.numpy as jnp
from jax import lax
from jax.experimental import pallas as pl
from jax.experimental.pallas import tpu as pltpu
from jax.experimental.pallas import tpu_sc as plsc

N = 10000
NP = 10240
E = 160000
EP = 163840
NG = 64
NTILE = 16
EPT = EP // NTILE
NROW = NP // NTILE
CHUNK = 160
N_CH = EPT // CHUNK
HALF = 128

_SC_MESH = dict(core_axis_name="c", subcore_axis_name="s")


def _make_deg():
    mesh = plsc.VectorSubcoreMesh(**_SC_MESH)

    @functools.partial(
        pl.kernel, mesh=mesh,
        out_type=jax.ShapeDtypeStruct((NP,), jnp.float32),
        scratch_types=[
            pltpu.VMEM_SHARED((NP,), jnp.float32),
            pltpu.VMEM((EPT,), jnp.int32),
            pltpu.VMEM((EPT,), jnp.float32),
            pltpu.VMEM((NROW,), jnp.float32),
        ],
    )
    def deg_k(dst_hbm, ones_hbm, deg_out, acc_sh, dst_v, ones_v, blk_v):
        c = lax.axis_index("c")
        s = lax.axis_index("s")
        pltpu.sync_copy(dst_hbm.at[pl.ds(s * EPT, EPT)], dst_v)
        pltpu.sync_copy(ones_hbm, ones_v)

        def zero_blk(i, carry):
            blk_v[pl.ds(i * 16, 16)] = jnp.zeros((16,), jnp.float32)
            return carry
        lax.fori_loop(0, NROW // 16, zero_blk, 0)
        pltpu.sync_copy(blk_v, acc_sh.at[pl.ds(s * NROW, NROW)])
        plsc.subcore_barrier()
        pltpu.sync_copy(ones_v, acc_sh.at[dst_v], add=True)
        plsc.subcore_barrier()

        @pl.when(c == 0)
        def _():
            pltpu.sync_copy(acc_sh.at[pl.ds(s * NROW, NROW)],
                            deg_out.at[pl.ds(s * NROW, NROW)])

    return deg_k


def _make_agg():
    mesh = plsc.VectorSubcoreMesh(**_SC_MESH)

    @functools.partial(
        pl.kernel, mesh=mesh,
        out_type=(jax.ShapeDtypeStruct((NP, HALF), jnp.float32),
                  jax.ShapeDtypeStruct((NP, HALF), jnp.float32)),
        scratch_types=[
            pltpu.VMEM_SHARED((NP, HALF), jnp.float32),
            pltpu.VMEM((CHUNK,), jnp.int32),
            pltpu.VMEM((CHUNK,), jnp.int32),
            pltpu.VMEM((CHUNK,), jnp.int32),
            pltpu.VMEM((CHUNK,), jnp.int32),
            pltpu.VMEM((CHUNK, HALF), jnp.float32),
            pltpu.VMEM((CHUNK, HALF), jnp.float32),
            pltpu.SemaphoreType.DMA,
            pltpu.SemaphoreType.DMA,
            pltpu.SemaphoreType.DMA,
            pltpu.SemaphoreType.DMA,
            pltpu.SemaphoreType.DMA,
            pltpu.SemaphoreType.DMA,
        ],
    )
    def agg_k(lo_hbm, hi_hbm, src_hbm, dst_hbm, out_lo, out_hi,
              acc_sh, srcc0, srcc1, dstc0, dstc1, rows0, rows1,
              si0, si1, sg0, sg1, ss0, ss1):
        c = lax.axis_index("c")
        s = lax.axis_index("s")
        slots = ((srcc0, dstc0, rows0, si0, sg0, ss0),
                 (srcc1, dstc1, rows1, si1, sg1, ss1))

        def run(hw_hbm, out_hbm):
            pltpu.sync_copy(hw_hbm.at[pl.ds(s * NROW, NROW)],
                            acc_sh.at[pl.ds(s * NROW, NROW)])
            plsc.subcore_barrier()
            base = s * EPT

            def issue_idx(k, slot):
                sc, dc, _, si, _, _ = slot
                off = pl.multiple_of(base + k * CHUNK, CHUNK)
                pltpu.async_copy(src_hbm.at[pl.ds(off, CHUNK)], sc, si)
                pltpu.async_copy(dst_hbm.at[pl.ds(off, CHUNK)], dc, si)

            def wait_idx(slot):
                sc, dc, _, si, _, _ = slot
                off = pl.multiple_of(base, CHUNK)
                pltpu.make_async_copy(src_hbm.at[pl.ds(off, CHUNK)], sc, si).wait()
                pltpu.make_async_copy(dst_hbm.at[pl.ds(off, CHUNK)], dc, si).wait()

            issue_idx(0, slots[0])
            wait_idx(slots[0])
            pltpu.async_copy(hw_hbm.at[slots[0][0]], slots[0][2], slots[0][4])

            def body2(i, carry):
                for b in (0, 1):
                    k = 2 * i + b
                    sc, dc, rv, si, sg, ss = slots[b]
                    scn, dcn, rvn, sin, sgn, ssn = slots[1 - b]

                    @pl.when(k >= 1)
                    def _():
                        pltpu.make_async_copy(
                            rvn, acc_sh.at[dcn], ssn).wait()

                    @pl.when(k + 1 < N_CH)
                    def _():
                        issue_idx(k + 1, slots[1 - b])

                    pltpu.make_async_copy(hw_hbm.at[sc], rv, sg).wait()

                    @pl.when(k + 1 < N_CH)
                    def _():
                        wait_idx(slots[1 - b])
                        pltpu.async_copy(hw_hbm.at[scn], rvn, sgn)

                    pltpu.async_copy(rv, acc_sh.at[dc], ss, add=True)
                return carry
            lax.fori_loop(0, N_CH // 2, body2, 0)
            pltpu.make_async_copy(rows1, acc_sh.at[dstc1], ss1).wait()
            plsc.subcore_barrier()
            pltpu.sync_copy(acc_sh.at[pl.ds(s * NROW, NROW)],
                            out_hbm.at[pl.ds(s * NROW, NROW)])

        @pl.when(c == 0)
        def _():
            run(lo_hbm, out_lo)

        @pl.when(c == 1)
        def _():
            run(hi_hbm, out_hi)

    return agg_k


_BLK = 512
_BLK3 = 400


def _tc1_body(x_ref, w1_ref, b1_ref, b2_ref, wa_ref, wg1_ref,
              deg_ref, lo_ref, hi_ref, dinv_ref):
    xb = x_ref[...]
    h1 = lax.dot_general(xb, w1_ref[...], (((1,), (1,)), ((), ())),
                         preferred_element_type=jnp.float32) + b1_ref[...]
    a1 = jnp.sum(h1 * wa_ref[...], axis=1, keepdims=True)
    a2 = jnp.sum(b2_ref[...] * wa_ref[...])
    w0 = jax.nn.sigmoid(a1 - a2)
    fused = w0 * h1 + (1.0 - w0) * b2_ref[...]
    hw = lax.dot_general(fused, wg1_ref[...], (((1,), (1,)), ((), ())),
                         preferred_element_type=jnp.float32)
    dinv = lax.rsqrt(deg_ref[...] + 1.0)
    hws = hw * dinv
    lo_ref[...] = hws[:, :HALF]
    hi_ref[...] = hws[:, HALF:]
    dinv_ref[...] = dinv


def _tc1(x_pad, W1, b1r, b2r, Wa, Wg1, deg2):
    grid = NP // _BLK
    return pl.pallas_call(
        _tc1_body,
        grid=(grid,),
        in_specs=[
            pl.BlockSpec((_BLK, 768), lambda i: (i, 0)),
            pl.BlockSpec((768, 768), lambda i: (0, 0)),
            pl.BlockSpec((1, 768), lambda i: (0, 0)),
            pl.BlockSpec((1, 768), lambda i: (0, 0)),
            pl.BlockSpec((1, 768), lambda i: (0, 0)),
            pl.BlockSpec((256, 768), lambda i: (0, 0)),
            pl.BlockSpec((_BLK, 1), lambda i: (i, 0)),
        ],
        out_specs=[
            pl.BlockSpec((_BLK, HALF), lambda i: (i, 0)),
            pl.BlockSpec((_BLK, HALF), lambda i: (i, 0)),
            pl.BlockSpec((_BLK, 1), lambda i: (i, 0)),
        ],
        out_shape=[
            jax.ShapeDtypeStruct((NP, HALF), jnp.float32),
            jax.ShapeDtypeStruct((NP, HALF), jnp.float32),
            jax.ShapeDtypeStruct((NP, 1), jnp.float32),
        ],
    )(x_pad, W1, b1r, b2r, Wa, Wg1, deg2)


def _tc2_body(lo_ref, hi_ref, dinv_ref, bg1_ref, wg2_ref, lo_o, hi_o):
    acc = jnp.concatenate([lo_ref[...], hi_ref[...]], axis=1)
    dinv = dinv_ref[...]
    h = jnp.maximum(acc * dinv + bg1_ref[...], 0.0)
    hw = lax.dot_general(h, wg2_ref[...], (((1,), (1,)), ((), ())),
                         preferred_element_type=jnp.float32) * dinv
    lo_o[...] = hw[:, :HALF]
    hi_o[...] = hw[:, HALF:]


def _tc2(acc_lo, acc_hi, dinv2, bg1r, Wg2):
    grid = NP // _BLK
    return pl.pallas_call(
        _tc2_body,
        grid=(grid,),
        in_specs=[
            pl.BlockSpec((_BLK, HALF), lambda i: (i, 0)),
            pl.BlockSpec((_BLK, HALF), lambda i: (i, 0)),
            pl.BlockSpec((_BLK, 1), lambda i: (i, 0)),
            pl.BlockSpec((1, 256), lambda i: (0, 0)),
            pl.BlockSpec((256, 256), lambda i: (0, 0)),
        ],
        out_specs=[
            pl.BlockSpec((_BLK, HALF), lambda i: (i, 0)),
            pl.BlockSpec((_BLK, HALF), lambda i: (i, 0)),
        ],
        out_shape=[
            jax.ShapeDtypeStruct((NP, HALF), jnp.float32),
            jax.ShapeDtypeStruct((NP, HALF), jnp.float32),
        ],
    )(acc_lo, acc_hi, dinv2, bg1r, Wg2)


def _tc3_body(lo_ref, hi_ref, dinv_ref, bg2_ref, batch_ref, wc1_ref, bc1_ref,
              wc2_ref, bc2_ref, out_ref, psum, pcnt):
    i = pl.program_id(0)

    @pl.when(i == 0)
    def _():
        psum[...] = jnp.zeros_like(psum)
        pcnt[...] = jnp.zeros_like(pcnt)

    acc = jnp.concatenate([lo_ref[...], hi_ref[...]], axis=1)
    h = jnp.maximum(acc * dinv_ref[...] + bg2_ref[...], 0.0)
    b = batch_ref[0]
    oh = (lax.broadcasted_iota(jnp.int32, (NG, _BLK3), 0) == b
          ).astype(jnp.float32)
    psum[...] += lax.dot_general(oh, h, (((1,), (0,)), ((), ())),
                                 preferred_element_type=jnp.float32)
    pcnt[...] += jnp.sum(oh, axis=1, keepdims=True)

    @pl.when(i == (N // _BLK3) - 1)
    def _():
        pooled = psum[...] / jnp.maximum(pcnt[...], 1.0)
        z = jnp.maximum(
            lax.dot_general(pooled, wc1_ref[...], (((1,), (1,)), ((), ())),
                            preferred_element_type=jnp.float32) + bc1_ref[...],
            0.0)
        out_ref[...] = lax.dot_general(
            z, wc2_ref[...], (((1,), (1,)), ((), ())),
            preferred_element_type=jnp.float32) + bc2_ref[...]


def _tc3(acc_lo, acc_hi, dinv2, bg2r, batch3, Wc1, bc1r, Wc2, bc2r):
    grid = N // _BLK3
    return pl.pallas_call(
        _tc3_body,
        grid=(grid,),
        in_specs=[
            pl.BlockSpec((_BLK3, HALF), lambda i: (i, 0)),
            pl.BlockSpec((_BLK3, HALF), lambda i: (i, 0)),
            pl.BlockSpec((_BLK3, 1), lambda i: (i, 0)),
            pl.BlockSpec((1, 256), lambda i: (0, 0)),
            pl.BlockSpec((1, 1, _BLK3), lambda i: (i, 0, 0)),
            pl.BlockSpec((128, 256), lambda i: (0, 0)),
            pl.BlockSpec((1, 128), lambda i: (0, 0)),
            pl.BlockSpec((5, 128), lambda i: (0, 0)),
            pl.BlockSpec((1, 5), lambda i: (0, 0)),
        ],
        out_specs=pl.BlockSpec((NG, 5), lambda i: (0, 0)),
        out_shape=jax.ShapeDtypeStruct((NG, 5), jnp.float32),
        scratch_shapes=[
            pltpu.VMEM((NG, 256), jnp.float32),
            pltpu.VMEM((NG, 1), jnp.float32),
        ],
    )(acc_lo, acc_hi, dinv2, bg2r, batch3, Wc1, bc1r, Wc2, bc2r)


_deg_k = _make_deg()
_agg_k = _make_agg()


def kernel(x, edge_index, batch, W1, b1, W2, b2, Wa, ba, Wg1, bg1, Wg2, bg2,
           Wc1, bc1, Wc2, bc2):
    del W2, ba
    f32 = jnp.float32
    src = edge_index[0]
    dst = edge_index[1]
    pad_e = jnp.full((EP - E,), N, jnp.int32)
    srcp = jnp.concatenate([src, pad_e])
    dstp = jnp.concatenate([dst, pad_e])
    ones = jnp.ones((EPT,), f32)
    x_pad = jnp.concatenate([x, jnp.zeros((NP - N, 768), f32)])
    b1r = b1.reshape(1, 768)
    b2r = b2.reshape(1, 768)
    bg1r = bg1.reshape(1, 256)
    bg2r = bg2.reshape(1, 256)
    bc1r = bc1.reshape(1, 128)
    bc2r = bc2.reshape(1, 5)
    batch3 = batch.reshape(N // _BLK3, 1, _BLK3)

    deg = _deg_k(dstp, ones)
    deg2 = deg.reshape(NP, 1)
    hw_lo, hw_hi, dinv2 = _tc1(x_pad, W1, b1r, b2r, Wa, Wg1, deg2)
    a1_lo, a1_hi = _agg_k(hw_lo, hw_hi, srcp, dstp)
    hw2_lo, hw2_hi = _tc2(a1_lo, a1_hi, dinv2, bg1r, Wg2)
    a2_lo, a2_hi = _agg_k(hw2_lo, hw2_hi, srcp, dstp)
    return _tc3(a2_lo, a2_hi, dinv2, bg2r, batch3, Wc1, bc1r, Wc2, bc2r)


# speedup vs baseline: 8.3931x; 1.0011x over previous
import functools

import jax
import jax.numpy as jnp
from jax import lax
from jax.experimental import pallas as pl
from jax.experimental.pallas import tpu as pltpu
from jax.experimental.pallas import tpu_sc as plsc

N = 10000
NP = 10240
E = 160000
EP = 163840
NG = 64
NTILE = 16
EPT = EP // NTILE
NROW = NP // NTILE
CHUNK = 160
N_CH = EPT // CHUNK
HALF = 128

_SC_MESH = dict(core_axis_name="c", subcore_axis_name="s")


def _make_deg():
    mesh = plsc.VectorSubcoreMesh(**_SC_MESH)

    @functools.partial(
        pl.kernel, mesh=mesh,
        out_type=jax.ShapeDtypeStruct((NP,), jnp.float32),
        scratch_types=[
            pltpu.VMEM_SHARED((NP,), jnp.float32),
            pltpu.VMEM((EPT,), jnp.int32),
            pltpu.VMEM((EPT,), jnp.float32),
            pltpu.VMEM((NROW,), jnp.float32),
        ],
    )
    def deg_k(dst_hbm, ones_hbm, deg_out, acc_sh, dst_v, ones_v, blk_v):
        c = lax.axis_index("c")
        s = lax.axis_index("s")
        pltpu.sync_copy(dst_hbm.at[pl.ds(s * EPT, EPT)], dst_v)
        pltpu.sync_copy(ones_hbm, ones_v)

        def zero_blk(i, carry):
            blk_v[pl.ds(i * 16, 16)] = jnp.zeros((16,), jnp.float32)
            return carry
        lax.fori_loop(0, NROW // 16, zero_blk, 0)
        pltpu.sync_copy(blk_v, acc_sh.at[pl.ds(s * NROW, NROW)])
        plsc.subcore_barrier()
        pltpu.sync_copy(ones_v, acc_sh.at[dst_v], add=True)
        plsc.subcore_barrier()

        @pl.when(c == 0)
        def _():
            pltpu.sync_copy(acc_sh.at[pl.ds(s * NROW, NROW)],
                            deg_out.at[pl.ds(s * NROW, NROW)])

    return deg_k


def _make_agg():
    mesh = plsc.VectorSubcoreMesh(**_SC_MESH)

    @functools.partial(
        pl.kernel, mesh=mesh,
        out_type=(jax.ShapeDtypeStruct((NP, HALF), jnp.float32),
                  jax.ShapeDtypeStruct((NP, HALF), jnp.float32)),
        scratch_types=[
            pltpu.VMEM_SHARED((NP, HALF), jnp.float32),
            pltpu.VMEM((CHUNK // 2,), jnp.int32),
            pltpu.VMEM((CHUNK // 2,), jnp.int32),
            pltpu.VMEM((CHUNK // 2,), jnp.int32),
            pltpu.VMEM((CHUNK // 2,), jnp.int32),
            pltpu.VMEM((CHUNK,), jnp.int32),
            pltpu.VMEM((CHUNK,), jnp.int32),
            pltpu.VMEM((CHUNK, HALF), jnp.float32),
            pltpu.VMEM((CHUNK, HALF), jnp.float32),
            pltpu.SemaphoreType.DMA,
            pltpu.SemaphoreType.DMA,
            pltpu.SemaphoreType.DMA,
            pltpu.SemaphoreType.DMA,
            pltpu.SemaphoreType.DMA,
            pltpu.SemaphoreType.DMA,
        ],
    )
    def agg_k(lo_hbm, hi_hbm, src_hbm, dst_hbm, out_lo, out_hi,
              acc_sh, sca0, scb0, sca1, scb1, dstc0, dstc1, rows0, rows1,
              si0, si1, sg0, sg1, ss0, ss1):
        c = lax.axis_index("c")
        s = lax.axis_index("s")
        HC = CHUNK // 2
        slots = ((sca0, scb0, dstc0, rows0, si0, sg0, ss0),
                 (sca1, scb1, dstc1, rows1, si1, sg1, ss1))

        def run(hw_hbm, out_hbm):
            pltpu.sync_copy(hw_hbm.at[pl.ds(s * NROW, NROW)],
                            acc_sh.at[pl.ds(s * NROW, NROW)])
            plsc.subcore_barrier()
            base = s * EPT

            def issue_idx(k, slot):
                sa, sb, dc, _, si, _, _ = slot
                off = pl.multiple_of(base + k * CHUNK, CHUNK)
                off2 = pl.multiple_of(base + k * CHUNK + HC, HC)
                pltpu.async_copy(src_hbm.at[pl.ds(off, HC)], sa, si)
                pltpu.async_copy(src_hbm.at[pl.ds(off2, HC)], sb, si)
                pltpu.async_copy(dst_hbm.at[pl.ds(off, CHUNK)], dc, si)

            def wait_idx(slot):
                sa, sb, dc, _, si, _, _ = slot
                off = pl.multiple_of(base, CHUNK)
                pltpu.make_async_copy(src_hbm.at[pl.ds(off, HC)], sa, si).wait()
                pltpu.make_async_copy(src_hbm.at[pl.ds(off, HC)], sb, si).wait()
                pltpu.make_async_copy(dst_hbm.at[pl.ds(off, CHUNK)], dc, si).wait()

            def issue_gather(hw_hbm, slot):
                sa, sb, _, rv, _, sg, _ = slot
                pltpu.async_copy(hw_hbm.at[sa], rv.at[pl.ds(0, HC)], sg)
                pltpu.async_copy(hw_hbm.at[sb], rv.at[pl.ds(HC, HC)], sg)

            def wait_gather(hw_hbm, slot):
                sa, sb, _, rv, _, sg, _ = slot
                pltpu.make_async_copy(
                    hw_hbm.at[sa], rv.at[pl.ds(0, HC)], sg).wait()
                pltpu.make_async_copy(
                    hw_hbm.at[sb], rv.at[pl.ds(HC, HC)], sg).wait()

            issue_idx(0, slots[0])
            wait_idx(slots[0])
            issue_gather(hw_hbm, slots[0])

            def body2(i, carry):
                for b in (0, 1):
                    k = 2 * i + b
                    _, _, dc, rv, si, sg, ss = slots[b]
                    _, _, dcn, rvn, sin, sgn, ssn = slots[1 - b]

                    @pl.when(k >= 1)
                    def _():
                        pltpu.make_async_copy(
                            rvn, acc_sh.at[dcn], ssn).wait()

                    @pl.when(k + 1 < N_CH)
                    def _():
                        issue_idx(k + 1, slots[1 - b])

                    wait_gather(hw_hbm, slots[b])

                    @pl.when(k + 1 < N_CH)
                    def _():
                        wait_idx(slots[1 - b])
                        issue_gather(hw_hbm, slots[1 - b])

                    pltpu.async_copy(rv, acc_sh.at[dc], ss, add=True)
                return carry
            lax.fori_loop(0, N_CH // 2, body2, 0)
            pltpu.make_async_copy(rows1, acc_sh.at[dstc1], ss1).wait()
            plsc.subcore_barrier()
            pltpu.sync_copy(acc_sh.at[pl.ds(s * NROW, NROW)],
                            out_hbm.at[pl.ds(s * NROW, NROW)])

        @pl.when(c == 0)
        def _():
            run(lo_hbm, out_lo)

        @pl.when(c == 1)
        def _():
            run(hi_hbm, out_hi)

    return agg_k


_BLK = 512
_BLK3 = 400


def _tc1_body(x_ref, w1_ref, b1_ref, b2_ref, wa_ref, wg1_ref,
              deg_ref, lo_ref, hi_ref, dinv_ref):
    xb = x_ref[...]
    h1 = lax.dot_general(xb, w1_ref[...], (((1,), (1,)), ((), ())),
                         preferred_element_type=jnp.float32) + b1_ref[...]
    a1 = jnp.sum(h1 * wa_ref[...], axis=1, keepdims=True)
    a2 = jnp.sum(b2_ref[...] * wa_ref[...])
    w0 = jax.nn.sigmoid(a1 - a2)
    fused = w0 * h1 + (1.0 - w0) * b2_ref[...]
    hw = lax.dot_general(fused, wg1_ref[...], (((1,), (1,)), ((), ())),
                         preferred_element_type=jnp.float32)
    dinv = lax.rsqrt(deg_ref[...] + 1.0)
    hws = hw * dinv
    lo_ref[...] = hws[:, :HALF]
    hi_ref[...] = hws[:, HALF:]
    dinv_ref[...] = dinv


def _tc1(x_pad, W1, b1r, b2r, Wa, Wg1, deg2):
    grid = NP // _BLK
    return pl.pallas_call(
        _tc1_body,
        grid=(grid,),
        in_specs=[
            pl.BlockSpec((_BLK, 768), lambda i: (i, 0)),
            pl.BlockSpec((768, 768), lambda i: (0, 0)),
            pl.BlockSpec((1, 768), lambda i: (0, 0)),
            pl.BlockSpec((1, 768), lambda i: (0, 0)),
            pl.BlockSpec((1, 768), lambda i: (0, 0)),
            pl.BlockSpec((256, 768), lambda i: (0, 0)),
            pl.BlockSpec((_BLK, 1), lambda i: (i, 0)),
        ],
        out_specs=[
            pl.BlockSpec((_BLK, HALF), lambda i: (i, 0)),
            pl.BlockSpec((_BLK, HALF), lambda i: (i, 0)),
            pl.BlockSpec((_BLK, 1), lambda i: (i, 0)),
        ],
        out_shape=[
            jax.ShapeDtypeStruct((NP, HALF), jnp.float32),
            jax.ShapeDtypeStruct((NP, HALF), jnp.float32),
            jax.ShapeDtypeStruct((NP, 1), jnp.float32),
        ],
    )(x_pad, W1, b1r, b2r, Wa, Wg1, deg2)


def _tc2_body(lo_ref, hi_ref, dinv_ref, bg1_ref, wg2_ref, lo_o, hi_o):
    acc = jnp.concatenate([lo_ref[...], hi_ref[...]], axis=1)
    dinv = dinv_ref[...]
    h = jnp.maximum(acc * dinv + bg1_ref[...], 0.0)
    hw = lax.dot_general(h, wg2_ref[...], (((1,), (1,)), ((), ())),
                         preferred_element_type=jnp.float32) * dinv
    lo_o[...] = hw[:, :HALF]
    hi_o[...] = hw[:, HALF:]


def _tc2(acc_lo, acc_hi, dinv2, bg1r, Wg2):
    grid = NP // _BLK
    return pl.pallas_call(
        _tc2_body,
        grid=(grid,),
        in_specs=[
            pl.BlockSpec((_BLK, HALF), lambda i: (i, 0)),
            pl.BlockSpec((_BLK, HALF), lambda i: (i, 0)),
            pl.BlockSpec((_BLK, 1), lambda i: (i, 0)),
            pl.BlockSpec((1, 256), lambda i: (0, 0)),
            pl.BlockSpec((256, 256), lambda i: (0, 0)),
        ],
        out_specs=[
            pl.BlockSpec((_BLK, HALF), lambda i: (i, 0)),
            pl.BlockSpec((_BLK, HALF), lambda i: (i, 0)),
        ],
        out_shape=[
            jax.ShapeDtypeStruct((NP, HALF), jnp.float32),
            jax.ShapeDtypeStruct((NP, HALF), jnp.float32),
        ],
    )(acc_lo, acc_hi, dinv2, bg1r, Wg2)


def _tc3_body(lo_ref, hi_ref, dinv_ref, bg2_ref, batch_ref, wc1_ref, bc1_ref,
              wc2_ref, bc2_ref, out_ref, psum, pcnt):
    i = pl.program_id(0)

    @pl.when(i == 0)
    def _():
        psum[...] = jnp.zeros_like(psum)
        pcnt[...] = jnp.zeros_like(pcnt)

    acc = jnp.concatenate([lo_ref[...], hi_ref[...]], axis=1)
    h = jnp.maximum(acc * dinv_ref[...] + bg2_ref[...], 0.0)
    b = batch_ref[0]
    oh = (lax.broadcasted_iota(jnp.int32, (NG, _BLK3), 0) == b
          ).astype(jnp.float32)
    psum[...] += lax.dot_general(oh, h, (((1,), (0,)), ((), ())),
                                 preferred_element_type=jnp.float32)
    pcnt[...] += jnp.sum(oh, axis=1, keepdims=True)

    @pl.when(i == (N // _BLK3) - 1)
    def _():
        pooled = psum[...] / jnp.maximum(pcnt[...], 1.0)
        z = jnp.maximum(
            lax.dot_general(pooled, wc1_ref[...], (((1,), (1,)), ((), ())),
                            preferred_element_type=jnp.float32) + bc1_ref[...],
            0.0)
        out_ref[...] = lax.dot_general(
            z, wc2_ref[...], (((1,), (1,)), ((), ())),
            preferred_element_type=jnp.float32) + bc2_ref[...]


def _tc3(acc_lo, acc_hi, dinv2, bg2r, batch3, Wc1, bc1r, Wc2, bc2r):
    grid = N // _BLK3
    return pl.pallas_call(
        _tc3_body,
        grid=(grid,),
        in_specs=[
            pl.BlockSpec((_BLK3, HALF), lambda i: (i, 0)),
            pl.BlockSpec((_BLK3, HALF), lambda i: (i, 0)),
            pl.BlockSpec((_BLK3, 1), lambda i: (i, 0)),
            pl.BlockSpec((1, 256), lambda i: (0, 0)),
            pl.BlockSpec((1, 1, _BLK3), lambda i: (i, 0, 0)),
            pl.BlockSpec((128, 256), lambda i: (0, 0)),
            pl.BlockSpec((1, 128), lambda i: (0, 0)),
            pl.BlockSpec((5, 128), lambda i: (0, 0)),
            pl.BlockSpec((1, 5), lambda i: (0, 0)),
        ],
        out_specs=pl.BlockSpec((NG, 5), lambda i: (0, 0)),
        out_shape=jax.ShapeDtypeStruct((NG, 5), jnp.float32),
        scratch_shapes=[
            pltpu.VMEM((NG, 256), jnp.float32),
            pltpu.VMEM((NG, 1), jnp.float32),
        ],
    )(acc_lo, acc_hi, dinv2, bg2r, batch3, Wc1, bc1r, Wc2, bc2r)


_deg_k = _make_deg()
_agg_k = _make_agg()


def kernel(x, edge_index, batch, W1, b1, W2, b2, Wa, ba, Wg1, bg1, Wg2, bg2,
           Wc1, bc1, Wc2, bc2):
    del W2, ba
    f32 = jnp.float32
    src = edge_index[0]
    dst = edge_index[1]
    pad_e = jnp.full((EP - E,), N, jnp.int32)
    srcp = jnp.concatenate([src, pad_e])
    dstp = jnp.concatenate([dst, pad_e])
    ones = jnp.ones((EPT,), f32)
    x_pad = jnp.concatenate([x, jnp.zeros((NP - N, 768), f32)])
    b1r = b1.reshape(1, 768)
    b2r = b2.reshape(1, 768)
    bg1r = bg1.reshape(1, 256)
    bg2r = bg2.reshape(1, 256)
    bc1r = bc1.reshape(1, 128)
    bc2r = bc2.reshape(1, 5)
    batch3 = batch.reshape(N // _BLK3, 1, _BLK3)

    deg = _deg_k(dstp, ones)
    deg2 = deg.reshape(NP, 1)
    hw_lo, hw_hi, dinv2 = _tc1(x_pad, W1, b1r, b2r, Wa, Wg1, deg2)
    a1_lo, a1_hi = _agg_k(hw_lo, hw_hi, srcp, dstp)
    hw2_lo, hw2_hi = _tc2(a1_lo, a1_hi, dinv2, bg1r, Wg2)
    a2_lo, a2_hi = _agg_k(hw2_lo, hw2_hi, srcp, dstp)
    return _tc3(a2_lo, a2_hi, dinv2, bg2r, batch3, Wc1, bc1r, Wc2, bc2r)
